# baseline (speedup 1.0000x reference)
import functools

import jax
import jax.numpy as jnp
from jax import lax
from jax.experimental import pallas as pl
from jax.experimental.pallas import tpu as pltpu
from jax.experimental.pallas import tpu_sc as plsc

NUM_CODES = 8192
DIM = 256

TOK_TILE = 1024


def _argmin_body(z_ref, e2_ref, idx_ref, loss_ref):
    i = pl.program_id(0)

    zt = z_ref[...]
    e2t = e2_ref[...]
    sz = jnp.sum(zt ** 2, axis=1, keepdims=True)
    se = 0.25 * jnp.sum(e2t ** 2, axis=1)
    mm2 = lax.dot_general(zt, e2t, (((1,), (1,)), ((), ())),
                          preferred_element_type=jnp.float32)
    ST = 32
    seB = jnp.broadcast_to(se[None, :], (ST, NUM_CODES))

    LB = 128
    nb = NUM_CODES // LB
    lane_iota = lax.broadcasted_iota(jnp.int32, (ST, LB), 1)
    big = jnp.int32(2**31 - 1)
    mt_parts = []
    idx_parts = []
    for s in range(TOK_TILE // ST):
        szb = jnp.broadcast_to(
            lax.slice(sz, (s * ST, 0), ((s + 1) * ST, 1)), (ST, LB))
        acc_v = jnp.full((ST, LB), jnp.inf, jnp.float32)
        acc_a = jnp.zeros((ST, LB), jnp.int32)
        for a in range(nb):
            mk = lax.slice(mm2, (s * ST, a * LB), ((s + 1) * ST, (a + 1) * LB))
            sek = lax.slice(seB, (0, a * LB), (ST, (a + 1) * LB))
            dk = (szb + mk) + sek
            better = dk < acc_v
            acc_v = jnp.where(better, dk, acc_v)
            acc_a = jnp.where(better, jnp.int32(a), acc_a)
        mrow = jnp.min(acc_v, axis=1, keepdims=True)
        gidx = acc_a * LB + lane_iota
        cand = jnp.where(acc_v == mrow, gidx, big)
        idx_parts.append(jnp.min(cand, axis=1))
        mt_parts.append(mrow[:, 0])
    idxt = jnp.concatenate(idx_parts)
    tile_loss = jnp.sum(jnp.concatenate(mt_parts))

    idx_ref[...] = idxt

    @pl.when(i == 0)
    def _():
        loss_ref[0, 0] = tile_loss

    @pl.when(i > 0)
    def _():
        loss_ref[0, 0] = loss_ref[0, 0] + tile_loss


def _argmin_search(z_flat, emb2):
    n_tok = z_flat.shape[0]
    n_i = n_tok // TOK_TILE
    return pl.pallas_call(
        _argmin_body,
        grid=(n_i,),
        in_specs=[
            pl.BlockSpec((TOK_TILE, DIM), lambda i: (i, 0)),
            pl.BlockSpec((NUM_CODES, DIM), lambda i: (0, 0)),
        ],
        out_specs=[
            pl.BlockSpec((TOK_TILE,), lambda i: (i,)),
            pl.BlockSpec((1, 1), lambda i: (0, 0),
                         memory_space=pltpu.SMEM),
        ],
        out_shape=[
            jax.ShapeDtypeStruct((n_tok,), jnp.int32),
            jax.ShapeDtypeStruct((1, 1), jnp.float32),
        ],
        compiler_params=pltpu.CompilerParams(
            dimension_semantics=("arbitrary",),
        ),
    )(z_flat, emb2)


def _make_sc_gather(n_tok):
    info = plsc.get_sparse_core_info()
    nw = info.num_cores * info.num_subcores
    b_per_w = n_tok // nw

    mesh = plsc.VectorSubcoreMesh(core_axis_name="c", subcore_axis_name="s")

    @functools.partial(
        pl.kernel,
        mesh=mesh,
        out_type=jax.ShapeDtypeStruct((n_tok, DIM), jnp.float32),
        scratch_types=[
            pltpu.VMEM((b_per_w,), jnp.int32),
            pltpu.VMEM((b_per_w, DIM), jnp.float32),
            pltpu.SemaphoreType.DMA,
        ],
    )
    def gather_kernel(emb_hbm, idx_hbm, out_hbm, idx_v, rows_v, sem):
        wid = lax.axis_index("s") * info.num_cores + lax.axis_index("c")
        base = wid * b_per_w
        pltpu.sync_copy(idx_hbm.at[pl.ds(base, b_per_w)], idx_v)
        pltpu.async_copy(emb_hbm.at[idx_v], rows_v, sem).wait()
        pltpu.sync_copy(rows_v, out_hbm.at[pl.ds(base, b_per_w)])

    return gather_kernel


def kernel(z, emb_weight):
    B, D, H, W = z.shape
    z_flat = jnp.transpose(z, (0, 2, 3, 1)).reshape(-1, D)
    n_tok = z_flat.shape[0]
    half = n_tok // 2

    emb2 = -2.0 * emb_weight
    gather = _make_sc_gather(half)

    idx_a, loss_a = _argmin_search(z_flat[:half], emb2)
    zq_a = gather(emb_weight, idx_a)
    idx_b, loss_b = _argmin_search(z_flat[half:], emb2)
    zq_b = gather(emb_weight, idx_b)

    indices = jnp.concatenate([idx_a, idx_b])
    z_q_flat = jnp.concatenate([zq_a, zq_b])

    z_q = z_q_flat.reshape(B, H, W, D)
    z_q = jnp.transpose(z_q, (0, 3, 1, 2))
    commitment_loss = ((loss_a[0, 0] + loss_b[0, 0])
                       / jnp.float32(B * D * H * W)).reshape(())
    z_q_st = z + lax.stop_gradient(z_q - z)
    indices_grid = indices.reshape(B, H, W)
    return (z_q_st, commitment_loss, indices_grid)


# speedup vs baseline: 1.1292x; 1.1292x over previous
import functools

import jax
import jax.numpy as jnp
from jax import lax
from jax.experimental import pallas as pl
from jax.experimental.pallas import tpu as pltpu
from jax.experimental.pallas import tpu_sc as plsc

NUM_CODES = 8192
DIM = 256

TOK_TILE = 1024


def _argmin_body(z_ref, e2_ref, idx_ref, loss_ref):
    i = pl.program_id(0)

    zt = z_ref[...]
    e2t = e2_ref[...]
    sz = jnp.sum(zt ** 2, axis=1, keepdims=True)
    se = 0.25 * jnp.sum(e2t ** 2, axis=1)
    mm2 = lax.dot_general(zt, e2t, (((1,), (1,)), ((), ())),
                          preferred_element_type=jnp.float32)
    ST = 32
    seB = jnp.broadcast_to(se[None, :], (ST, NUM_CODES))

    LB = 128
    nb = NUM_CODES // LB
    lane_iota = lax.broadcasted_iota(jnp.int32, (ST, LB), 1)
    big = jnp.int32(2**31 - 1)
    mt_parts = []
    idx_parts = []
    for s in range(TOK_TILE // ST):
        szb = jnp.broadcast_to(
            lax.slice(sz, (s * ST, 0), ((s + 1) * ST, 1)), (ST, LB))
        acc_v = jnp.full((ST, LB), jnp.inf, jnp.float32)
        acc_a = jnp.zeros((ST, LB), jnp.int32)
        for a in range(nb):
            mk = lax.slice(mm2, (s * ST, a * LB), ((s + 1) * ST, (a + 1) * LB))
            sek = lax.slice(seB, (0, a * LB), (ST, (a + 1) * LB))
            dk = (szb + mk) + sek
            better = dk < acc_v
            acc_v = jnp.where(better, dk, acc_v)
            acc_a = jnp.where(better, jnp.int32(a), acc_a)
        mrow = jnp.min(acc_v, axis=1, keepdims=True)
        gidx = acc_a * LB + lane_iota
        cand = jnp.where(acc_v == mrow, gidx, big)
        idx_parts.append(jnp.min(cand, axis=1))
        mt_parts.append(mrow[:, 0])
    idxt = jnp.concatenate(idx_parts)
    tile_loss = jnp.sum(jnp.concatenate(mt_parts))

    idx_ref[...] = idxt

    @pl.when(i == 0)
    def _():
        loss_ref[0, 0] = tile_loss

    @pl.when(i > 0)
    def _():
        loss_ref[0, 0] = loss_ref[0, 0] + tile_loss


def _argmin_search(z_flat, emb2):
    n_tok = z_flat.shape[0]
    n_i = n_tok // TOK_TILE
    return pl.pallas_call(
        _argmin_body,
        grid=(n_i,),
        in_specs=[
            pl.BlockSpec((TOK_TILE, DIM), lambda i: (i, 0)),
            pl.BlockSpec((NUM_CODES, DIM), lambda i: (0, 0)),
        ],
        out_specs=[
            pl.BlockSpec((TOK_TILE,), lambda i: (i,)),
            pl.BlockSpec((1, 1), lambda i: (0, 0),
                         memory_space=pltpu.SMEM),
        ],
        out_shape=[
            jax.ShapeDtypeStruct((n_tok,), jnp.int32),
            jax.ShapeDtypeStruct((1, 1), jnp.float32),
        ],
        compiler_params=pltpu.CompilerParams(
            dimension_semantics=("arbitrary",),
        ),
    )(z_flat, emb2)


def _make_sc_gather(n_tok):
    info = plsc.get_sparse_core_info()
    nw = info.num_cores * info.num_subcores
    b_per_w = n_tok // nw

    mesh = plsc.VectorSubcoreMesh(core_axis_name="c", subcore_axis_name="s")

    @functools.partial(
        pl.kernel,
        mesh=mesh,
        out_type=jax.ShapeDtypeStruct((n_tok, DIM), jnp.float32),
        scratch_types=[
            pltpu.VMEM((b_per_w,), jnp.int32),
            pltpu.VMEM((b_per_w, DIM), jnp.float32),
            pltpu.SemaphoreType.DMA,
        ],
    )
    def gather_kernel(emb_hbm, idx_hbm, out_hbm, idx_v, rows_v, sem):
        wid = lax.axis_index("s") * info.num_cores + lax.axis_index("c")
        base = wid * b_per_w
        pltpu.sync_copy(idx_hbm.at[pl.ds(base, b_per_w)], idx_v)
        pltpu.async_copy(emb_hbm.at[idx_v], rows_v, sem).wait()
        pltpu.sync_copy(rows_v, out_hbm.at[pl.ds(base, b_per_w)])

    return gather_kernel


def kernel(z, emb_weight):
    B, D, H, W = z.shape
    z_flat = jnp.transpose(z, (0, 2, 3, 1)).reshape(-1, D)
    n_tok = z_flat.shape[0]

    emb2 = -2.0 * emb_weight
    indices, loss_sum = _argmin_search(z_flat, emb2)
    z_q_flat = _make_sc_gather(n_tok)(emb_weight, indices)

    z_q = z_q_flat.reshape(B, H, W, D)
    z_q = jnp.transpose(z_q, (0, 3, 1, 2))
    commitment_loss = (loss_sum[0, 0]
                       / jnp.float32(B * D * H * W)).reshape(())
    z_q_st = z + lax.stop_gradient(z_q - z)
    indices_grid = indices.reshape(B, H, W)
    return (z_q_st, commitment_loss, indices_grid)


# speedup vs baseline: 1.1662x; 1.0328x over previous
import functools

import jax
import jax.numpy as jnp
from jax import lax
from jax.experimental import pallas as pl
from jax.experimental.pallas import tpu as pltpu
from jax.experimental.pallas import tpu_sc as plsc

NUM_CODES = 8192
DIM = 256

TOK_TILE = 1024


def _argmin_body(z_ref, e2_ref, idx_ref, loss_ref, seB_ref):
    i = pl.program_id(0)
    ST = 32

    zt = z_ref[...]
    e2t = e2_ref[...]
    sz = jnp.sum(zt ** 2, axis=1, keepdims=True)

    @pl.when(i == 0)
    def _():
        se = 0.25 * jnp.sum(e2t ** 2, axis=1)
        seB_ref[...] = jnp.broadcast_to(se[None, :], (ST, NUM_CODES))

    mm2 = lax.dot_general(zt, e2t, (((1,), (1,)), ((), ())),
                          preferred_element_type=jnp.float32)
    seB = seB_ref[...]

    LB = 128
    nb = NUM_CODES // LB
    lane_iota = lax.broadcasted_iota(jnp.int32, (ST, LB), 1)
    big = jnp.int32(2**31 - 1)
    mt_parts = []
    idx_parts = []
    for s in range(TOK_TILE // ST):
        szb = jnp.broadcast_to(
            lax.slice(sz, (s * ST, 0), ((s + 1) * ST, 1)), (ST, LB))
        acc_v = jnp.full((ST, LB), jnp.inf, jnp.float32)
        acc_a = jnp.zeros((ST, LB), jnp.int32)
        for a in range(nb):
            mk = lax.slice(mm2, (s * ST, a * LB), ((s + 1) * ST, (a + 1) * LB))
            sek = lax.slice(seB, (0, a * LB), (ST, (a + 1) * LB))
            dk = (szb + mk) + sek
            better = dk < acc_v
            acc_v = jnp.where(better, dk, acc_v)
            acc_a = jnp.where(better, jnp.int32(a), acc_a)
        mrow = jnp.min(acc_v, axis=1, keepdims=True)
        gidx = acc_a * LB + lane_iota
        cand = jnp.where(acc_v == mrow, gidx, big)
        idx_parts.append(jnp.min(cand, axis=1))
        mt_parts.append(mrow[:, 0])
    idxt = jnp.concatenate(idx_parts)
    tile_loss = jnp.sum(jnp.concatenate(mt_parts))

    idx_ref[...] = idxt

    @pl.when(i == 0)
    def _():
        loss_ref[0, 0] = tile_loss

    @pl.when(i > 0)
    def _():
        loss_ref[0, 0] = loss_ref[0, 0] + tile_loss


def _argmin_search(z_flat, emb2):
    n_tok = z_flat.shape[0]
    n_i = n_tok // TOK_TILE
    return pl.pallas_call(
        _argmin_body,
        grid=(n_i,),
        in_specs=[
            pl.BlockSpec((TOK_TILE, DIM), lambda i: (i, 0)),
            pl.BlockSpec((NUM_CODES, DIM), lambda i: (0, 0)),
        ],
        out_specs=[
            pl.BlockSpec((TOK_TILE,), lambda i: (i,)),
            pl.BlockSpec((1, 1), lambda i: (0, 0),
                         memory_space=pltpu.SMEM),
        ],
        out_shape=[
            jax.ShapeDtypeStruct((n_tok,), jnp.int32),
            jax.ShapeDtypeStruct((1, 1), jnp.float32),
        ],
        scratch_shapes=[
            pltpu.VMEM((32, NUM_CODES), jnp.float32),
        ],
        compiler_params=pltpu.CompilerParams(
            dimension_semantics=("arbitrary",),
        ),
    )(z_flat, emb2)


def _make_sc_gather(n_tok):
    info = plsc.get_sparse_core_info()
    nw = info.num_cores * info.num_subcores
    b_per_w = n_tok // nw

    mesh = plsc.VectorSubcoreMesh(core_axis_name="c", subcore_axis_name="s")

    @functools.partial(
        pl.kernel,
        mesh=mesh,
        out_type=jax.ShapeDtypeStruct((n_tok, DIM), jnp.float32),
        scratch_types=[
            pltpu.VMEM((b_per_w,), jnp.int32),
            pltpu.VMEM((b_per_w, DIM), jnp.float32),
            pltpu.SemaphoreType.DMA,
        ],
    )
    def gather_kernel(emb_hbm, idx_hbm, out_hbm, idx_v, rows_v, sem):
        wid = lax.axis_index("s") * info.num_cores + lax.axis_index("c")
        base = wid * b_per_w
        pltpu.sync_copy(idx_hbm.at[pl.ds(base, b_per_w)], idx_v)
        pltpu.async_copy(emb_hbm.at[idx_v], rows_v, sem).wait()
        pltpu.sync_copy(rows_v, out_hbm.at[pl.ds(base, b_per_w)])

    return gather_kernel


def kernel(z, emb_weight):
    B, D, H, W = z.shape
    z_flat = jnp.transpose(z, (0, 2, 3, 1)).reshape(-1, D)
    n_tok = z_flat.shape[0]

    emb2 = -2.0 * emb_weight
    indices, loss_sum = _argmin_search(z_flat, emb2)
    z_q_flat = _make_sc_gather(n_tok)(emb_weight, indices)

    z_q = z_q_flat.reshape(B, H, W, D)
    z_q = jnp.transpose(z_q, (0, 3, 1, 2))
    commitment_loss = (loss_sum[0, 0]
                       / jnp.float32(B * D * H * W)).reshape(())
    z_q_st = z + lax.stop_gradient(z_q - z)
    indices_grid = indices.reshape(B, H, W)
    return (z_q_st, commitment_loss, indices_grid)
